# baseline (speedup 1.0000x reference)
import jax
import jax.numpy as jnp
from jax.experimental import pallas as pl
from jax.experimental.pallas import tpu as pltpu

B, D, H = 16384, 4096, 7
BLOCK_B = 256


def _gating_kernel(f1_ref, f2_ref, w1_ref, b1_ref, w2_ref, o_ref):
    f1 = f1_ref[...]
    f2 = f2_ref[...]
    w1 = w1_ref[...]
    b1 = b1_ref[...]
    w2 = w2_ref[...]

    h1 = jnp.maximum(jnp.dot(f1, w1, preferred_element_type=jnp.float32) + b1, 0.0)
    h2 = jnp.maximum(jnp.dot(f2, w1, preferred_element_type=jnp.float32) + b1, 0.0)
    g1 = jnp.dot(h1, w2, preferred_element_type=jnp.float32)
    g2 = jnp.dot(h2, w2, preferred_element_type=jnp.float32)
    d = (jnp.sum(g1, axis=1, keepdims=True)
         - jnp.sum(g2, axis=1, keepdims=True)) * (1.0 / H)
    a1 = jax.nn.sigmoid(d)
    o_ref[...] = a1 * f1 + (1.0 - a1) * f2


@jax.jit
def kernel(features1, features2, W1, b1, W2, b2):
    del b2
    grid = (B // BLOCK_B,)
    return pl.pallas_call(
        _gating_kernel,
        grid=grid,
        in_specs=[
            pl.BlockSpec((BLOCK_B, D), lambda i: (i, 0)),
            pl.BlockSpec((BLOCK_B, D), lambda i: (i, 0)),
            pl.BlockSpec((D, H), lambda i: (0, 0)),
            pl.BlockSpec((1, H), lambda i: (0, 0)),
            pl.BlockSpec((H, H), lambda i: (0, 0)),
        ],
        out_specs=pl.BlockSpec((BLOCK_B, D), lambda i: (i, 0)),
        out_shape=jax.ShapeDtypeStruct((B, D), jnp.float32),
        compiler_params=pltpu.CompilerParams(
            dimension_semantics=("parallel",),
            vmem_limit_bytes=56 * 1024 * 1024,
        ),
    )(features1, features2, W1, b1.reshape(1, H), W2)


# speedup vs baseline: 1.0393x; 1.0393x over previous
import jax
import jax.numpy as jnp
from jax.experimental import pallas as pl
from jax.experimental.pallas import tpu as pltpu

B, D, H = 16384, 4096, 7
BLOCK_B = 512


def _gating_kernel(f1_ref, f2_ref, w1_ref, b1_ref, w2_ref, o_ref):
    f1 = f1_ref[...]
    f2 = f2_ref[...]
    w1 = w1_ref[...]
    b1 = b1_ref[...]
    w2 = w2_ref[...]

    h1 = jnp.maximum(jnp.dot(f1, w1, preferred_element_type=jnp.float32) + b1, 0.0)
    h2 = jnp.maximum(jnp.dot(f2, w1, preferred_element_type=jnp.float32) + b1, 0.0)
    g1 = jnp.dot(h1, w2, preferred_element_type=jnp.float32)
    g2 = jnp.dot(h2, w2, preferred_element_type=jnp.float32)
    d = (jnp.sum(g1, axis=1, keepdims=True)
         - jnp.sum(g2, axis=1, keepdims=True)) * (1.0 / H)
    a1 = jax.nn.sigmoid(d)
    o_ref[...] = a1 * f1 + (1.0 - a1) * f2


@jax.jit
def kernel(features1, features2, W1, b1, W2, b2):
    del b2
    grid = (B // BLOCK_B,)
    return pl.pallas_call(
        _gating_kernel,
        grid=grid,
        in_specs=[
            pl.BlockSpec((BLOCK_B, D), lambda i: (i, 0)),
            pl.BlockSpec((BLOCK_B, D), lambda i: (i, 0)),
            pl.BlockSpec((D, H), lambda i: (0, 0)),
            pl.BlockSpec((1, H), lambda i: (0, 0)),
            pl.BlockSpec((H, H), lambda i: (0, 0)),
        ],
        out_specs=pl.BlockSpec((BLOCK_B, D), lambda i: (i, 0)),
        out_shape=jax.ShapeDtypeStruct((B, D), jnp.float32),
        compiler_params=pltpu.CompilerParams(
            dimension_semantics=("parallel",),
            vmem_limit_bytes=56 * 1024 * 1024,
        ),
    )(features1, features2, W1, b1.reshape(1, H), W2)
